# baseline (speedup 1.0000x reference)
import functools

import jax
import jax.numpy as jnp
from jax import lax
from jax.experimental import pallas as pl
from jax.experimental.pallas import tpu as pltpu
from jax.experimental.pallas import tpu_sc as plsc

B = 16384
D = 16
PACK = 8
W = D * PACK
NC = 2
NS = 16
NW = NC * NS
BPW = B // NW
NCHUNK = 2
CHUNK = BPW // NCHUNK

_mesh = plsc.VectorSubcoreMesh(core_axis_name="c", subcore_axis_name="s")


@functools.partial(
    pl.kernel,
    mesh=_mesh,
    compiler_params=pltpu.CompilerParams(needs_layout_passes=False),
    out_type=jax.ShapeDtypeStruct((B,), jnp.float32),
    scratch_types=[
        pltpu.VMEM((BPW,), jnp.int32),
        pltpu.VMEM((BPW,), jnp.int32),
        pltpu.VMEM((CHUNK,), jnp.int32),
        pltpu.VMEM((CHUNK,), jnp.int32),
        pltpu.VMEM((CHUNK, W), jnp.float32),
        pltpu.VMEM((CHUNK, W), jnp.float32),
        pltpu.VMEM((BPW,), jnp.float32),
        pltpu.SemaphoreType.DMA,
        pltpu.SemaphoreType.DMA,
    ],
)
def _mf_sc(xu_hbm, xi_hbm, user_hbm, item_hbm, out_hbm,
           xu_v, xi_v, qu_v, qi_v, urows_v, irows_v, out_v, sem_u, sem_i):
    wid = lax.axis_index("s") * NC + lax.axis_index("c")
    base = wid * BPW
    pltpu.sync_copy(xu_hbm.at[pl.ds(base, BPW)], xu_v)
    pltpu.sync_copy(xi_hbm.at[pl.ds(base, BPW)], xi_v)
    lanes = lax.iota(jnp.int32, D)

    for c in range(NCHUNK):
        c0 = c * CHUNK
        for k in range(CHUNK // D):
            qu_v[pl.ds(k * D, D)] = xu_v[pl.ds(c0 + k * D, D)] >> 3
            qi_v[pl.ds(k * D, D)] = xi_v[pl.ds(c0 + k * D, D)] >> 3
        cu = pltpu.async_copy(user_hbm.at[qu_v], urows_v, sem_u)
        ci = pltpu.async_copy(item_hbm.at[qi_v], irows_v, sem_i)
        cu.wait()
        ci.wait()

        def body(blk, carry, c0=c0):
            row0 = blk * D
            rows = row0 + lanes
            uoff = (xu_v[pl.ds(c0 + row0, D)] & 7) * D
            ioff = (xi_v[pl.ds(c0 + row0, D)] & 7) * D
            acc = jnp.zeros((D,), jnp.float32)
            for d in range(D):
                uc = plsc.load_gather(urows_v, [rows, uoff + d])
                vc = plsc.load_gather(irows_v, [rows, ioff + d])
                acc = acc + uc * vc
            out_v[pl.ds(c0 + row0, D)] = acc
            return carry

        lax.fori_loop(0, CHUNK // D, body, 0)

    pltpu.sync_copy(out_v, out_hbm.at[pl.ds(base, BPW)])


def kernel(X, user_emb, item_emb):
    xu = X[:, 0]
    xi = X[:, 1]
    u2 = user_emb.reshape(-1, W)
    i2 = item_emb.reshape(-1, W)
    out = _mf_sc(xu, xi, u2, i2)
    return out.reshape(B, 1)


# speedup vs baseline: 6.9785x; 6.9785x over previous
import functools

import jax
import jax.numpy as jnp
from jax import lax
from jax.experimental import pallas as pl
from jax.experimental.pallas import tpu as pltpu
from jax.experimental.pallas import tpu_sc as plsc

B = 16384
D = 16
NC = 2
NS = 16
NW = NC * NS
BPW = B // NW
NBLK = BPW // D
RING = 16
W = 128
V = 1000000
TAIL0 = (V // W) * W
TAILBASE = V - W

_mesh = plsc.VectorSubcoreMesh(core_axis_name="c", subcore_axis_name="s")


@functools.partial(
    pl.kernel,
    mesh=_mesh,
    compiler_params=pltpu.CompilerParams(needs_layout_passes=False),
    out_type=jax.ShapeDtypeStruct((B,), jnp.float32),
    scratch_types=[
        pltpu.VMEM((BPW,), jnp.int32),
        pltpu.VMEM((BPW,), jnp.int32),
        pltpu.VMEM((RING, D, W), jnp.float32),
        pltpu.VMEM((RING, D, W), jnp.float32),
        pltpu.VMEM((D, W), jnp.float32),
        pltpu.VMEM((D, W), jnp.float32),
        pltpu.VMEM((BPW,), jnp.float32),
        pltpu.SemaphoreType.DMA,
        pltpu.SemaphoreType.DMA,
    ],
)
def _mf_sc(xu_hbm, xi_hbm, ut_hbm, it_hbm, tu_hbm, ti_hbm, out_hbm,
           xu_v, xi_v, ubufs, ibufs, tubuf, tibuf, out_v, sem_u, sem_i):
    wid = lax.axis_index("s") * NC + lax.axis_index("c")
    base = wid * BPW
    pltpu.sync_copy(xu_hbm.at[pl.ds(base, BPW)], xu_v)
    pltpu.sync_copy(xi_hbm.at[pl.ds(base, BPW)], xi_v)
    pltpu.sync_copy(tu_hbm, tubuf)
    pltpu.sync_copy(ti_hbm, tibuf)

    lanes = lax.iota(jnp.int32, D)

    def fire(uvec, ivec, l):
        iu = jnp.minimum(uvec[l], TAILBASE)
        ii = jnp.minimum(ivec[l], TAILBASE)
        off_u = pl.multiple_of(iu & ~(W - 1), W)
        off_i = pl.multiple_of(ii & ~(W - 1), W)
        pltpu.async_copy(ut_hbm.at[:, pl.ds(off_u, W)], ubufs.at[l], sem_u)
        pltpu.async_copy(it_hbm.at[:, pl.ds(off_i, W)], ibufs.at[l], sem_i)

    def col(bufs, tbuf, l, i):
        c_blk = jnp.minimum(i - (jnp.minimum(i, TAILBASE) & ~(W - 1)), W - 1)
        c_tail = jnp.clip(i - TAILBASE, 0, W - 1)
        v_blk = plsc.load_gather(bufs.at[l], [lanes, jnp.full((D,), c_blk, jnp.int32)])
        v_tail = plsc.load_gather(tbuf, [lanes, jnp.full((D,), c_tail, jnp.int32)])
        return jnp.where(jnp.full((D,), i >= TAIL0), v_tail, v_blk)

    def consume(uvec, ivec, l, acc):
        pltpu.make_async_copy(ut_hbm.at[:, pl.ds(0, W)], ubufs.at[l], sem_u).wait()
        pltpu.make_async_copy(it_hbm.at[:, pl.ds(0, W)], ibufs.at[l], sem_i).wait()
        ucol = col(ubufs, tubuf, l, uvec[l])
        icol = col(ibufs, tibuf, l, ivec[l])
        s = jnp.sum(ucol * icol)
        return jnp.where(lanes == l, s, acc)

    def id_vecs(b):
        return xu_v[pl.ds(b * D, D)], xi_v[pl.ds(b * D, D)]

    u0, i0 = id_vecs(0)
    for l in range(D):
        fire(u0, i0, l)

    def body(b, carry):
        ub, ib = id_vecs(b)
        ub1, ib1 = id_vecs(b + 1)
        acc = jnp.zeros((D,), jnp.float32)
        for l in range(D):
            acc = consume(ub, ib, l, acc)
            fire(ub1, ib1, l)
        out_v[pl.ds(b * D, D)] = acc
        return carry

    lax.fori_loop(0, NBLK - 1, body, 0)

    ul, il = id_vecs(NBLK - 1)
    acc = jnp.zeros((D,), jnp.float32)
    for l in range(D):
        acc = consume(ul, il, l, acc)
    out_v[pl.ds((NBLK - 1) * D, D)] = acc

    pltpu.sync_copy(out_v, out_hbm.at[pl.ds(base, BPW)])


def kernel(X, user_emb, item_emb):
    xu = X[:, 0]
    xi = X[:, 1]
    ut = user_emb.T
    it = item_emb.T
    tu = user_emb[TAILBASE:].T
    ti = item_emb[TAILBASE:].T
    out = _mf_sc(xu, xi, ut, it, tu, ti)
    return out.reshape(B, 1)
